# baseline (speedup 1.0000x reference)
import functools

import jax
import jax.numpy as jnp
from jax import lax
from jax.experimental import pallas as pl
from jax.experimental.pallas import tpu as pltpu
from jax.experimental.pallas import tpu_sc as plsc

Z = 128
BATCH = 16384

_info = plsc.get_sparse_core_info()
_NC, _NS, _L = _info.num_cores, _info.num_subcores, _info.num_lanes
_NW = _NC * _NS
_BPW = BATCH // _NW
_C = 128
_NCHUNK = _BPW // _C

_mesh = plsc.VectorSubcoreMesh(core_axis_name="c", subcore_axis_name="s")

_SPLAT_DNUMS = lax.GatherDimensionNumbers(
    offset_dims=(), collapsed_slice_dims=(0,), start_index_map=(0,))


@functools.partial(
    pl.kernel,
    mesh=_mesh,
    out_type=jax.ShapeDtypeStruct((BATCH, Z), jnp.float32),
    scratch_types=[
        pltpu.VMEM((_BPW,), jnp.int32),
        pltpu.VMEM((_BPW,), jnp.float32),
        pltpu.VMEM((_C, Z), jnp.float32),
        pltpu.VMEM((_C, Z), jnp.float32),
        pltpu.SemaphoreType.DMA,
    ],
)
def _sc_fma_gather(z_hbm, idx_hbm, s_hbm, tab_hbm, out_hbm,
                   idx_v, s_v, rows_v, zb_v, sem):
    wid = lax.axis_index("s") * _NC + lax.axis_index("c")
    base = wid * _BPW
    pltpu.sync_copy(idx_hbm.at[pl.ds(base, _BPW)], idx_v)
    pltpu.sync_copy(s_hbm.at[pl.ds(base, _BPW)], s_v)
    for k in range(_NCHUNK):
        off = k * _C
        gat = pltpu.async_copy(tab_hbm.at[idx_v.at[pl.ds(off, _C)]], rows_v, sem)
        pltpu.sync_copy(z_hbm.at[pl.ds(base + off, _C)], zb_v)
        gat.wait()

        def body(g, carry):
            sv16 = s_v[pl.ds(off + g * _L, _L)]
            for j in range(_L):
                splat = lax.gather(
                    sv16, jnp.full((_L, 1), j, jnp.int32),
                    _SPLAT_DNUMS, (1,),
                    mode=lax.GatherScatterMode.PROMISE_IN_BOUNDS)
                b = g * _L + j
                for c in range(Z // _L):
                    sl = pl.ds(c * _L, _L)
                    zb_v[b, sl] = zb_v[b, sl] + rows_v[b, sl] * splat
            return carry

        lax.fori_loop(0, _C // _L, body, 0)
        pltpu.sync_copy(zb_v, out_hbm.at[pl.ds(base + off, _C)])


def kernel(z, labels, a):
    idx = labels[0].astype(jnp.int32)
    scale = labels[1]
    table = a.T
    return _sc_fma_gather(z, idx, scale, table)


# speedup vs baseline: 1.2522x; 1.2522x over previous
import functools

import jax
import jax.numpy as jnp
from jax import lax
from jax.experimental import pallas as pl
from jax.experimental.pallas import tpu as pltpu
from jax.experimental.pallas import tpu_sc as plsc

Z = 128
BATCH = 16384

_info = plsc.get_sparse_core_info()
_NC, _NS, _L = _info.num_cores, _info.num_subcores, _info.num_lanes
_NW = _NC * _NS
_BPW = BATCH // _NW
_C = 128
_NCHUNK = _BPW // _C
_NBUF = 3

_mesh = plsc.VectorSubcoreMesh(core_axis_name="c", subcore_axis_name="s")

_SPLAT_DNUMS = lax.GatherDimensionNumbers(
    offset_dims=(), collapsed_slice_dims=(0,), start_index_map=(0,))


@functools.partial(
    pl.kernel,
    mesh=_mesh,
    out_type=jax.ShapeDtypeStruct((BATCH, Z), jnp.float32),
    scratch_types=(
        [pltpu.VMEM((_BPW,), jnp.int32),
         pltpu.VMEM((_BPW,), jnp.float32)]
        + [pltpu.VMEM((_C, Z), jnp.float32)] * _NBUF
        + [pltpu.VMEM((_C, Z), jnp.float32)] * _NBUF
        + [pltpu.SemaphoreType.DMA] * (2 * _NBUF)
    ),
)
def _sc_fma_gather(z_hbm, idx_hbm, s_hbm, tab_hbm, out_hbm,
                   idx_v, s_v, r0, r1, r2, y0, y1, y2,
                   gi0, gi1, gi2, go0, go1, go2):
    rows = (r0, r1, r2)
    ybuf = (y0, y1, y2)
    isem = (gi0, gi1, gi2)
    osem = (go0, go1, go2)
    wid = lax.axis_index("s") * _NC + lax.axis_index("c")
    base = wid * _BPW
    pltpu.sync_copy(idx_hbm.at[pl.ds(base, _BPW)], idx_v)
    pltpu.sync_copy(s_hbm.at[pl.ds(base, _BPW)], s_v)

    gat = [None] * _NCHUNK
    zcp = [None] * _NCHUNK
    ost = [None] * _NCHUNK

    def start(k):
        j = k % _NBUF
        off = k * _C
        gat[k] = pltpu.async_copy(
            tab_hbm.at[idx_v.at[pl.ds(off, _C)]], rows[j], isem[j])
        zcp[k] = pltpu.async_copy(
            z_hbm.at[pl.ds(base + off, _C)], ybuf[j], isem[j])

    def compute(k):
        j = k % _NBUF
        off = k * _C
        rj, yj = rows[j], ybuf[j]

        def body(g, carry):
            sv16 = s_v[pl.ds(off + g * _L, _L)]
            for jj in range(_L):
                splat = lax.gather(
                    sv16, jnp.full((_L, 1), jj, jnp.int32),
                    _SPLAT_DNUMS, (1,),
                    mode=lax.GatherScatterMode.PROMISE_IN_BOUNDS)
                b = g * _L + jj
                for c in range(Z // _L):
                    sl = pl.ds(c * _L, _L)
                    plsc.addupdate(yj.at[b, sl], rj[b, sl] * splat)
            return carry

        lax.fori_loop(0, _C // _L, body, 0)

    for k in range(min(_NBUF, _NCHUNK)):
        start(k)
    for k in range(_NCHUNK):
        j = k % _NBUF
        gat[k].wait()
        zcp[k].wait()
        compute(k)
        ost[k] = pltpu.async_copy(
            ybuf[j], out_hbm.at[pl.ds(base + k * _C, _C)], osem[j])
        nxt = k + _NBUF - 1
        if k >= 1 and nxt < _NCHUNK:
            ost[k - 1].wait()
            start(nxt)
    for k in range(max(0, _NCHUNK - _NBUF + 1), _NCHUNK):
        ost[k].wait()


def kernel(z, labels, a):
    idx = labels[0].astype(jnp.int32)
    scale = labels[1]
    table = a.T
    return _sc_fma_gather(z, idx, scale, table)
